# baseline (speedup 1.0000x reference)
import functools

import jax
import jax.numpy as jnp
from jax import lax
from jax.experimental import pallas as pl
from jax.experimental.pallas import tpu as pltpu
from jax.experimental.pallas import tpu_sc as plsc

_COMMITMENT_COST = 0.5


def _vq_body(z_ref, emb_ref, idx_ref, fidx_ref, loss_ref, *,
             num_heads, head_dim, num_codes):
    z = z_ref[...]
    idx_cols = []
    fidx_cols = []
    total = jnp.zeros((), jnp.float32)
    for h in range(num_heads):
        zh = z[:, h * head_dim:(h + 1) * head_dim]
        eh = emb_ref[h]
        prod = jnp.dot(zh, eh.T, preferred_element_type=jnp.float32)
        zsq = jnp.sum(zh * zh, axis=1, keepdims=True)
        csq = jnp.sum(eh * eh, axis=1)
        dist = zsq + csq[None, :] - 2.0 * prod
        minv = jnp.min(dist, axis=1, keepdims=True)
        iota_f = jax.lax.broadcasted_iota(jnp.int32, dist.shape, 1).astype(jnp.float32)
        idx_f = jnp.min(jnp.where(dist == minv, iota_f, float(num_codes)),
                        axis=1, keepdims=True)
        idx_cols.append(idx_f)
        fidx_cols.append(idx_f + float(h * num_codes))
        total = total + jnp.sum(minv)

    idx_ref[...] = jnp.concatenate(idx_cols, axis=1).astype(jnp.int32)
    fidx_ref[...] = jnp.concatenate(fidx_cols, axis=1).astype(jnp.int32)
    total2d = total.reshape(1, 1)

    @pl.when(pl.program_id(0) == 0)
    def _init():
        loss_ref[...] = total2d

    @pl.when(pl.program_id(0) != 0)
    def _acc():
        loss_ref[...] += total2d


def _make_sc_gather(num_rows, row_dim, num_workers, chunk):
    rows_per_w = num_rows // num_workers
    n_chunks = rows_per_w // chunk
    mesh = plsc.VectorSubcoreMesh(core_axis_name="c", subcore_axis_name="s")

    @functools.partial(
        pl.kernel,
        mesh=mesh,
        out_type=jax.ShapeDtypeStruct((num_rows, row_dim), jnp.float32),
        scratch_types=[
            pltpu.VMEM((chunk,), jnp.int32),
            pltpu.VMEM((chunk, row_dim), jnp.float32),
            pltpu.SemaphoreType.DMA,
        ],
    )
    def sc_gather(table_hbm, fidx_hbm, out_hbm, idx_v, rows_v, sem):
        num_cores = lax.axis_size("c")
        wid = lax.axis_index("s") * num_cores + lax.axis_index("c")
        base = wid * rows_per_w

        def body(c, carry):
            off = pl.multiple_of(base + c * chunk, 8)
            pltpu.sync_copy(fidx_hbm.at[pl.ds(off, chunk)], idx_v)
            pltpu.async_copy(table_hbm.at[idx_v], rows_v, sem).wait()
            pltpu.sync_copy(rows_v, out_hbm.at[pl.ds(off, chunk)])
            return carry

        lax.fori_loop(0, n_chunks, body, 0)

    return sc_gather


def kernel(inputs, embeddings):
    B, T, D = inputs.shape
    H, K, hd = embeddings.shape
    N = B * T
    flat = inputs.reshape(N, D)

    Tb = min(2048, N)
    grid = (N // Tb,)

    body = functools.partial(_vq_body, num_heads=H, head_dim=hd, num_codes=K)
    idx, fidx, loss_sum = pl.pallas_call(
        body,
        grid=grid,
        in_specs=[
            pl.BlockSpec((Tb, D), lambda i: (i, 0)),
            pl.BlockSpec((H, K, hd), lambda i: (0, 0, 0)),
        ],
        out_specs=[
            pl.BlockSpec((Tb, H), lambda i: (i, 0)),
            pl.BlockSpec((Tb, H), lambda i: (i, 0)),
            pl.BlockSpec((1, 1), lambda i: (0, 0)),
        ],
        out_shape=[
            jax.ShapeDtypeStruct((N, H), jnp.int32),
            jax.ShapeDtypeStruct((N, H), jnp.int32),
            jax.ShapeDtypeStruct((1, 1), jnp.float32),
        ],
    )(flat, embeddings)

    table = embeddings.reshape(H * K, hd)
    sc_gather = _make_sc_gather(N * H, hd, num_workers=32, chunk=128)
    q = sc_gather(table, fidx.reshape(N * H))

    loss = loss_sum[0, 0] * (1.0 + _COMMITMENT_COST) / (N * D)
    return (q.reshape(B, T, D), loss, idx)


# speedup vs baseline: 1.0067x; 1.0067x over previous
import functools

import jax
import jax.numpy as jnp
from jax import lax
from jax.experimental import pallas as pl
from jax.experimental.pallas import tpu as pltpu
from jax.experimental.pallas import tpu_sc as plsc

_COMMITMENT_COST = 0.5


def _vq_body(z_ref, emb_ref, idx_ref, fidx_ref, loss_ref, *,
             num_heads, head_dim, num_codes):
    z = z_ref[...]
    idx_cols = []
    fidx_cols = []
    total = jnp.zeros((), jnp.float32)
    for h in range(num_heads):
        zh = z[:, h * head_dim:(h + 1) * head_dim]
        eh = emb_ref[h]
        prod = jnp.dot(zh, eh.T, preferred_element_type=jnp.float32)
        zsq = jnp.sum(zh * zh, axis=1, keepdims=True)
        csq = jnp.sum(eh * eh, axis=1)
        dist = zsq + csq[None, :] - 2.0 * prod
        minv = jnp.min(dist, axis=1, keepdims=True)
        iota_f = jax.lax.broadcasted_iota(jnp.int32, dist.shape, 1).astype(jnp.float32)
        idx_f = jnp.min(jnp.where(dist == minv, iota_f, float(num_codes)),
                        axis=1, keepdims=True)
        idx_cols.append(idx_f)
        fidx_cols.append(idx_f + float(h * num_codes))
        total = total + jnp.sum(minv)

    idx_ref[...] = jnp.concatenate(idx_cols, axis=1).astype(jnp.int32)
    fidx_ref[...] = jnp.concatenate(fidx_cols, axis=1).astype(jnp.int32)
    total2d = total.reshape(1, 1)

    @pl.when(pl.program_id(0) == 0)
    def _init():
        loss_ref[...] = total2d

    @pl.when(pl.program_id(0) != 0)
    def _acc():
        loss_ref[...] += total2d


def _make_sc_gather(num_rows, row_dim, num_workers, chunk, nbuf=4, lookahead=2):
    rows_per_w = num_rows // num_workers
    n = rows_per_w // chunk
    assert lookahead < nbuf and n > nbuf
    assert (n - lookahead - lookahead) % nbuf == 0
    mesh = plsc.VectorSubcoreMesh(core_axis_name="c", subcore_axis_name="s")

    row_buf_types = [pltpu.VMEM((chunk, row_dim), jnp.float32)] * nbuf
    sem_types = [pltpu.SemaphoreType.DMA] * (2 * nbuf)

    @functools.partial(
        pl.kernel,
        mesh=mesh,
        out_type=jax.ShapeDtypeStruct((num_rows, row_dim), jnp.float32),
        scratch_types=[pltpu.VMEM((n, chunk), jnp.int32)] + row_buf_types + sem_types,
    )
    def sc_gather(table_hbm, fidx_hbm, out_hbm, idx_all, *bufs_and_sems):
        rows = bufs_and_sems[:nbuf]
        gsem = bufs_and_sems[nbuf:2 * nbuf]
        wsem = bufs_and_sems[2 * nbuf:]
        num_cores = lax.axis_size("c")
        wid = lax.axis_index("s") * num_cores + lax.axis_index("c")
        base = wid * rows_per_w

        pltpu.sync_copy(fidx_hbm.at[pl.ds(wid * n, n)], idx_all)

        def start_gather(c, b):
            pltpu.async_copy(table_hbm.at[idx_all.at[c]], rows[b], gsem[b])

        def start_write(c, b):
            off = pl.multiple_of(base + c * chunk, 8)
            pltpu.async_copy(rows[b], out_hbm.at[pl.ds(off, chunk)], wsem[b])

        for c in range(lookahead):
            start_gather(c, c % nbuf)

        for c in range(lookahead):
            start_gather(c + lookahead, (c + lookahead) % nbuf)
            pltpu.make_async_copy(table_hbm.at[idx_all.at[c]], rows[c % nbuf],
                                  gsem[c % nbuf]).wait()
            start_write(c, c % nbuf)

        steady = n - 2 * lookahead

        def body(o, carry):
            c0 = lookahead + o * nbuf
            for j in range(nbuf):
                c = c0 + j
                b = (lookahead + j) % nbuf
                bn = (b + lookahead) % nbuf
                pltpu.make_async_copy(rows[bn], out_hbm.at[pl.ds(0, chunk)],
                                      wsem[bn]).wait()
                start_gather(c + lookahead, bn)
                pltpu.make_async_copy(table_hbm.at[idx_all.at[c]], rows[b],
                                      gsem[b]).wait()
                start_write(c, b)
            return carry

        lax.fori_loop(0, steady // nbuf, body, 0)

        for k in range(lookahead):
            c = n - lookahead + k
            b = (n - lookahead + k) % nbuf
            pltpu.make_async_copy(table_hbm.at[idx_all.at[0]], rows[b],
                                  gsem[b]).wait()
            start_write(c, b)

        for b in range(nbuf):
            pltpu.make_async_copy(rows[b], out_hbm.at[pl.ds(0, chunk)],
                                  wsem[b]).wait()

    return sc_gather


def kernel(inputs, embeddings):
    B, T, D = inputs.shape
    H, K, hd = embeddings.shape
    N = B * T
    flat = inputs.reshape(N, D)

    Tb = min(2048, N)
    grid = (N // Tb,)

    body = functools.partial(_vq_body, num_heads=H, head_dim=hd, num_codes=K)
    idx, fidx, loss_sum = pl.pallas_call(
        body,
        grid=grid,
        in_specs=[
            pl.BlockSpec((Tb, D), lambda i: (i, 0)),
            pl.BlockSpec((H, K, hd), lambda i: (0, 0, 0)),
        ],
        out_specs=[
            pl.BlockSpec((Tb, H), lambda i: (i, 0)),
            pl.BlockSpec((Tb, H), lambda i: (i, 0)),
            pl.BlockSpec((1, 1), lambda i: (0, 0)),
        ],
        out_shape=[
            jax.ShapeDtypeStruct((N, H), jnp.int32),
            jax.ShapeDtypeStruct((N, H), jnp.int32),
            jax.ShapeDtypeStruct((1, 1), jnp.float32),
        ],
    )(flat, embeddings)

    table = embeddings.reshape(H * K, hd)
    chunk = 64
    sc_gather = _make_sc_gather(N * H, hd, num_workers=32, chunk=chunk)
    q = sc_gather(table, fidx.reshape((N * H) // chunk, chunk))

    loss = loss_sum[0, 0] * (1.0 + _COMMITMENT_COST) / (N * D)
    return (q.reshape(B, T, D), loss, idx)


# speedup vs baseline: 2.5600x; 2.5430x over previous
import functools

import jax
import jax.numpy as jnp
from jax.experimental import pallas as pl

_COMMITMENT_COST = 0.5


def _vq_body(z_ref, emb_ref, q_ref, idx_ref, loss_ref, *, num_heads, head_dim, num_codes):
    z = z_ref[...]
    q_cols = []
    idx_cols = []
    total = jnp.zeros((), jnp.float32)
    for h in range(num_heads):
        zh = z[:, h * head_dim:(h + 1) * head_dim]
        eh = emb_ref[h]
        prod = jnp.dot(zh, eh.T, preferred_element_type=jnp.float32)
        zsq = jnp.sum(zh * zh, axis=1, keepdims=True)
        csq = jnp.sum(eh * eh, axis=1)
        dist = zsq + csq[None, :] - 2.0 * prod
        minv = jnp.min(dist, axis=1, keepdims=True)
        iota_f = jax.lax.broadcasted_iota(jnp.int32, dist.shape, 1).astype(jnp.float32)
        idx_f = jnp.min(jnp.where(dist == minv, iota_f, float(num_codes)),
                        axis=1, keepdims=True)
        onehot = (iota_f == idx_f).astype(jnp.float32)
        qh = jnp.dot(onehot, eh, preferred_element_type=jnp.float32,
                     precision=jax.lax.Precision.HIGHEST)
        q_cols.append(qh)
        idx_cols.append(idx_f)
        total = total + jnp.sum(minv)

    q_ref[...] = jnp.concatenate(q_cols, axis=1)
    idx_ref[...] = jnp.concatenate(idx_cols, axis=1).astype(jnp.int32)
    total2d = total.reshape(1, 1)

    @pl.when(pl.program_id(0) == 0)
    def _init():
        loss_ref[...] = total2d

    @pl.when(pl.program_id(0) != 0)
    def _acc():
        loss_ref[...] += total2d


def kernel(inputs, embeddings):
    B, T, D = inputs.shape
    H, K, hd = embeddings.shape
    N = B * T
    flat = inputs.reshape(N, D)

    Tb = min(2048, N)
    grid = (N // Tb,)

    body = functools.partial(_vq_body, num_heads=H, head_dim=hd, num_codes=K)
    q, idx, loss_sum = pl.pallas_call(
        body,
        grid=grid,
        in_specs=[
            pl.BlockSpec((Tb, D), lambda i: (i, 0)),
            pl.BlockSpec((H, K, hd), lambda i: (0, 0, 0)),
        ],
        out_specs=[
            pl.BlockSpec((Tb, D), lambda i: (i, 0)),
            pl.BlockSpec((Tb, H), lambda i: (i, 0)),
            pl.BlockSpec((1, 1), lambda i: (0, 0)),
        ],
        out_shape=[
            jax.ShapeDtypeStruct((N, D), jnp.float32),
            jax.ShapeDtypeStruct((N, H), jnp.int32),
            jax.ShapeDtypeStruct((1, 1), jnp.float32),
        ],
    )(flat, embeddings)

    loss = loss_sum[0, 0] * (1.0 + _COMMITMENT_COST) / (N * D)
    return (q.reshape(B, T, D), loss, idx)


# speedup vs baseline: 4.0332x; 1.5755x over previous
import functools

import jax
import jax.numpy as jnp
from jax.experimental import pallas as pl

_COMMITMENT_COST = 0.5


def _vq_body(z_ref, emb_ref, q_ref, idx_ref, loss_ref, *, num_heads, head_dim, num_codes):
    z = z_ref[...]
    q_cols = []
    idx_cols = []
    total = jnp.zeros((), jnp.float32)
    for h in range(num_heads):
        zh = z[:, h * head_dim:(h + 1) * head_dim]
        eh = emb_ref[h]
        prod = jnp.dot(zh, eh.T, preferred_element_type=jnp.float32)
        zsq = jnp.sum(zh * zh, axis=1, keepdims=True)
        csq = jnp.sum(eh * eh, axis=1)
        dist = zsq + csq[None, :] - 2.0 * prod
        minv = jnp.min(dist, axis=1, keepdims=True)
        iota_f = jax.lax.broadcasted_iota(jnp.int32, dist.shape, 1).astype(jnp.float32)
        idx_f = jnp.min(jnp.where(dist == minv, iota_f, float(num_codes)),
                        axis=1, keepdims=True)
        onehot = (iota_f == idx_f).astype(jnp.float32)
        qh = jnp.dot(onehot, eh, preferred_element_type=jnp.float32)
        q_cols.append(qh)
        idx_cols.append(idx_f)
        total = total + jnp.sum(minv)

    q_ref[...] = jnp.concatenate(q_cols, axis=1)
    idx_ref[...] = jnp.concatenate(idx_cols, axis=1).astype(jnp.int32)
    total2d = total.reshape(1, 1)

    @pl.when(pl.program_id(0) == 0)
    def _init():
        loss_ref[...] = total2d

    @pl.when(pl.program_id(0) != 0)
    def _acc():
        loss_ref[...] += total2d


def kernel(inputs, embeddings):
    B, T, D = inputs.shape
    H, K, hd = embeddings.shape
    N = B * T
    flat = inputs.reshape(N, D)

    Tb = min(2048, N)
    grid = (N // Tb,)

    body = functools.partial(_vq_body, num_heads=H, head_dim=hd, num_codes=K)
    q, idx, loss_sum = pl.pallas_call(
        body,
        grid=grid,
        in_specs=[
            pl.BlockSpec((Tb, D), lambda i: (i, 0)),
            pl.BlockSpec((H, K, hd), lambda i: (0, 0, 0)),
        ],
        out_specs=[
            pl.BlockSpec((Tb, D), lambda i: (i, 0)),
            pl.BlockSpec((Tb, H), lambda i: (i, 0)),
            pl.BlockSpec((1, 1), lambda i: (0, 0)),
        ],
        out_shape=[
            jax.ShapeDtypeStruct((N, D), jnp.float32),
            jax.ShapeDtypeStruct((N, H), jnp.int32),
            jax.ShapeDtypeStruct((1, 1), jnp.float32),
        ],
    )(flat, embeddings)

    loss = loss_sum[0, 0] * (1.0 + _COMMITMENT_COST) / (N * D)
    return (q.reshape(B, T, D), loss, idx)
